# baseline (speedup 1.0000x reference)
import functools

import jax
import jax.numpy as jnp
import numpy as np
from jax import lax
from jax.experimental import pallas as pl
from jax.experimental.pallas import tpu as pltpu
from jax.experimental.pallas import tpu_sc as plsc

N_PTS = 2_000_000
RES = 256
SENTINEL = RES * RES * RES
LO = np.float32(0.0) + np.float32(1e-5)
HI = np.float32(1.0) - np.float32(1e-5)

NC, NS, L = 2, 16, 16
NW = NC * NS

C = 8000
N_CHUNKS = N_PTS // C
GROUPS = C // L
N_SLOTS = (N_CHUNKS + NW - 1) // NW
N_PAIRS = N_SLOTS // 2
assert N_SLOTS % 2 == 0

_mesh = plsc.VectorSubcoreMesh(core_axis_name="c", subcore_axis_name="s")

_f32 = jnp.float32
_scratch = [
    pltpu.VMEM((C,), _f32), pltpu.VMEM((C,), _f32),
    pltpu.VMEM((C,), _f32), pltpu.VMEM((C,), _f32),
    pltpu.VMEM((C,), _f32), pltpu.VMEM((C,), _f32),
    pltpu.VMEM((C,), jnp.int32), pltpu.VMEM((C,), jnp.int32),
    pltpu.VMEM((C,), jnp.bool_), pltpu.VMEM((C,), jnp.bool_),
    pltpu.SemaphoreType.DMA,
    pltpu.SemaphoreType.DMA,
]


@functools.partial(
    pl.kernel,
    out_type=jax.ShapeDtypeStruct((N_PTS,), jnp.bool_),
    mesh=_mesh,
    compiler_params=pltpu.CompilerParams(needs_layout_passes=False),
    scratch_types=_scratch,
)
def _occupancy_kernel(xyz_hbm, grid_hbm, out_hbm,
                      x0, x1, y0, y1, z0, z1, i0, i1, r0, r1,
                      in_sem, g_sem):
    wid = lax.axis_index("s") * NC + lax.axis_index("c")
    xb, yb, zb, ib, rb = (x0, x1), (y0, y1), (z0, z1), (i0, i1), (r0, r1)

    def slot_base(j):
        return (wid + j * NW) * C

    def slot_valid(j):
        return wid + j * NW < N_CHUNKS

    def start_in(j, b):
        base = slot_base(j)
        pltpu.async_copy(xyz_hbm.at[pl.ds(base, C)], xb[b], in_sem)
        pltpu.async_copy(xyz_hbm.at[pl.ds(N_PTS + base, C)], yb[b], in_sem)
        pltpu.async_copy(xyz_hbm.at[pl.ds(2 * N_PTS + base, C)], zb[b], in_sem)

    def wait_in(b):
        for buf in (xb[b], yb[b], zb[b]):
            pltpu.make_async_copy(xyz_hbm.at[pl.ds(0, C)], buf, in_sem).wait()

    def compute(b):
        x_v, y_v, z_v, idx_v = xb[b], yb[b], zb[b], ib[b]

        def grp(g, carry):
            s = pl.ds(g * L, L)
            x = x_v[s]
            y = y_v[s]
            z = z_v[s]
            ix = (x * np.float32(RES)).astype(jnp.int32)
            iy = (y * np.float32(RES)).astype(jnp.int32)
            iz = (z * np.float32(RES)).astype(jnp.int32)
            hi = jnp.maximum(jnp.maximum(x, y), z)
            lo = jnp.minimum(jnp.minimum(x, y), z)
            inv = (hi >= HI) | (lo < LO)
            idx = ix * (RES * RES) + iy * RES + iz
            idx_v[s] = jnp.where(inv, SENTINEL, idx)
            return carry

        lax.fori_loop(0, GROUPS, grp, 0, unroll=4)

    def start_gather(b):
        pltpu.async_copy(grid_hbm.at[ib[b]], rb[b], g_sem)

    def finish_gather(j, b):
        pltpu.make_async_copy(grid_hbm.at[ib[b]], rb[b], g_sem).wait()
        pltpu.sync_copy(rb[b], out_hbm.at[pl.ds(slot_base(j), C)])

    start_in(0, 0)

    def pair_body(p, carry):
        j0 = 2 * p
        j1 = 2 * p + 1
        wait_in(0)

        @pl.when(slot_valid(j1))
        def _():
            start_in(j1, 1)

        compute(0)

        @pl.when(p > 0)
        def _():
            finish_gather(j1 - 2, 1)

        start_gather(0)

        @pl.when(slot_valid(j1))
        def _():
            wait_in(1)

        @pl.when(p < N_PAIRS - 1)
        def _():
            start_in(j0 + 2, 0)

        @pl.when(slot_valid(j1))
        def _():
            compute(1)

        finish_gather(j0, 0)

        @pl.when(slot_valid(j1))
        def _():
            start_gather(1)

        return carry

    lax.fori_loop(0, N_PAIRS, pair_body, 0)

    @pl.when(slot_valid(N_SLOTS - 1))
    def _():
        finish_gather(N_SLOTS - 1, 1)


def kernel(pts, grid_flat):
    xyz = jnp.ravel(pts, order="F")
    return _occupancy_kernel(xyz, grid_flat)


# speedup vs baseline: 2.4795x; 2.4795x over previous
import functools

import jax
import jax.numpy as jnp
import numpy as np
from jax import lax
from jax.experimental import pallas as pl
from jax.experimental.pallas import tpu as pltpu
from jax.experimental.pallas import tpu_sc as plsc

N_PTS = 2_000_000
RES = 256
SENTINEL = RES * RES * RES
LO = np.float32(0.0) + np.float32(1e-5)
HI = np.float32(1.0) - np.float32(1e-5)

NC, NS, L = 2, 16, 16
NW = NC * NS

C = 8000
N_CHUNKS = N_PTS // C
GROUPS = C // L
N_SLOTS = (N_CHUNKS + NW - 1) // NW
N_PAIRS = N_SLOTS // 2
assert N_SLOTS % 2 == 0

_mesh = plsc.VectorSubcoreMesh(core_axis_name="c", subcore_axis_name="s")

_f32 = jnp.float32
_scratch = [
    pltpu.VMEM((C,), _f32), pltpu.VMEM((C,), _f32),
    pltpu.VMEM((C,), _f32), pltpu.VMEM((C,), _f32),
    pltpu.VMEM((C,), _f32), pltpu.VMEM((C,), _f32),
    pltpu.VMEM((C,), jnp.int32), pltpu.VMEM((C,), jnp.int32),
    pltpu.VMEM((C,), jnp.bool_), pltpu.VMEM((C,), jnp.bool_),
    pltpu.SemaphoreType.DMA,
    pltpu.SemaphoreType.DMA,
]


@functools.partial(
    pl.kernel,
    out_type=jax.ShapeDtypeStruct((N_PTS,), jnp.bool_),
    mesh=_mesh,
    compiler_params=pltpu.CompilerParams(needs_layout_passes=False),
    scratch_types=_scratch,
)
def _occupancy_kernel(x_hbm, y_hbm, z_hbm, grid_hbm, out_hbm,
                      x0, x1, y0, y1, z0, z1, i0, i1, r0, r1,
                      in_sem, g_sem):
    wid = lax.axis_index("s") * NC + lax.axis_index("c")
    xb, yb, zb, ib, rb = (x0, x1), (y0, y1), (z0, z1), (i0, i1), (r0, r1)

    def slot_base(j):
        return (wid + j * NW) * C

    def slot_valid(j):
        return wid + j * NW < N_CHUNKS

    def start_in(j, b):
        base = slot_base(j)
        pltpu.async_copy(x_hbm.at[pl.ds(base, C)], xb[b], in_sem)
        pltpu.async_copy(y_hbm.at[pl.ds(base, C)], yb[b], in_sem)
        pltpu.async_copy(z_hbm.at[pl.ds(base, C)], zb[b], in_sem)

    def wait_in(b):
        for buf in (xb[b], yb[b], zb[b]):
            pltpu.make_async_copy(x_hbm.at[pl.ds(0, C)], buf, in_sem).wait()

    def compute(b):
        x_v, y_v, z_v, idx_v = xb[b], yb[b], zb[b], ib[b]

        def grp(g, carry):
            s = pl.ds(g * L, L)
            x = x_v[s]
            y = y_v[s]
            z = z_v[s]
            ix = (x * np.float32(RES)).astype(jnp.int32)
            iy = (y * np.float32(RES)).astype(jnp.int32)
            iz = (z * np.float32(RES)).astype(jnp.int32)
            hi = jnp.maximum(jnp.maximum(x, y), z)
            lo = jnp.minimum(jnp.minimum(x, y), z)
            inv = (hi >= HI) | (lo < LO)
            idx = ix * (RES * RES) + iy * RES + iz
            idx_v[s] = jnp.where(inv, SENTINEL, idx)
            return carry

        lax.fori_loop(0, GROUPS, grp, 0, unroll=4)

    def start_gather(b):
        pltpu.async_copy(grid_hbm.at[ib[b]], rb[b], g_sem)

    def finish_gather(j, b):
        pltpu.make_async_copy(grid_hbm.at[ib[b]], rb[b], g_sem).wait()
        pltpu.sync_copy(rb[b], out_hbm.at[pl.ds(slot_base(j), C)])

    start_in(0, 0)

    def pair_body(p, carry):
        j0 = 2 * p
        j1 = 2 * p + 1
        wait_in(0)

        @pl.when(slot_valid(j1))
        def _():
            start_in(j1, 1)

        compute(0)

        @pl.when(p > 0)
        def _():
            finish_gather(j1 - 2, 1)

        start_gather(0)

        @pl.when(slot_valid(j1))
        def _():
            wait_in(1)

        @pl.when(p < N_PAIRS - 1)
        def _():
            start_in(j0 + 2, 0)

        @pl.when(slot_valid(j1))
        def _():
            compute(1)

        finish_gather(j0, 0)

        @pl.when(slot_valid(j1))
        def _():
            start_gather(1)

        return carry

    lax.fori_loop(0, N_PAIRS, pair_body, 0)

    @pl.when(slot_valid(N_SLOTS - 1))
    def _():
        finish_gather(N_SLOTS - 1, 1)


def kernel(pts, grid_flat):
    pts_t = pts.T
    x = pts_t[0]
    y = pts_t[1]
    z = pts_t[2]
    return _occupancy_kernel(x, y, z, grid_flat)


# speedup vs baseline: 2.4802x; 1.0003x over previous
import functools

import jax
import jax.numpy as jnp
import numpy as np
from jax import lax
from jax.experimental import pallas as pl
from jax.experimental.pallas import tpu as pltpu
from jax.experimental.pallas import tpu_sc as plsc

N_PTS = 2_000_000
RES = 256
SENTINEL = RES * RES * RES
LO = np.float32(0.0) + np.float32(1e-5)
HI = np.float32(1.0) - np.float32(1e-5)

NC, NS, L = 2, 16, 16
NW = NC * NS

C = 8000
N_CHUNKS = N_PTS // C
GROUPS = C // L
N_SLOTS = (N_CHUNKS + NW - 1) // NW
N_PAIRS = N_SLOTS // 2
assert N_SLOTS % 2 == 0

_mesh = plsc.VectorSubcoreMesh(core_axis_name="c", subcore_axis_name="s")

_f32 = jnp.float32
_scratch = [
    pltpu.VMEM((C,), _f32), pltpu.VMEM((C,), _f32),
    pltpu.VMEM((C,), _f32), pltpu.VMEM((C,), _f32),
    pltpu.VMEM((C,), _f32), pltpu.VMEM((C,), _f32),
    pltpu.VMEM((C,), jnp.int32), pltpu.VMEM((C,), jnp.int32),
    pltpu.VMEM((C,), jnp.bool_), pltpu.VMEM((C,), jnp.bool_),
    pltpu.SemaphoreType.DMA,
    pltpu.SemaphoreType.DMA,
]


@functools.partial(
    pl.kernel,
    out_type=jax.ShapeDtypeStruct((N_PTS,), jnp.bool_),
    mesh=_mesh,
    compiler_params=pltpu.CompilerParams(needs_layout_passes=False),
    scratch_types=_scratch,
)
def _occupancy_kernel(x_hbm, y_hbm, z_hbm, grid_hbm, out_hbm,
                      x0, x1, y0, y1, z0, z1, i0, i1, r0, r1,
                      in_sem, g_sem):
    wid = lax.axis_index("s") * NC + lax.axis_index("c")
    xb, yb, zb, ib, rb = (x0, x1), (y0, y1), (z0, z1), (i0, i1), (r0, r1)

    def slot_base(j):
        return (wid + j * NW) * C

    def slot_valid(j):
        return wid + j * NW < N_CHUNKS

    def start_in(j, b):
        base = slot_base(j)
        pltpu.async_copy(x_hbm.at[pl.ds(base, C)], xb[b], in_sem)
        pltpu.async_copy(y_hbm.at[pl.ds(base, C)], yb[b], in_sem)
        pltpu.async_copy(z_hbm.at[pl.ds(base, C)], zb[b], in_sem)

    def wait_in(b):
        for buf in (xb[b], yb[b], zb[b]):
            pltpu.make_async_copy(x_hbm.at[pl.ds(0, C)], buf, in_sem).wait()

    def compute(b):
        x_v, y_v, z_v, idx_v = xb[b], yb[b], zb[b], ib[b]

        def grp(g, carry):
            s = pl.ds(g * L, L)
            x = x_v[s]
            y = y_v[s]
            z = z_v[s]
            ix = (x * np.float32(RES)).astype(jnp.int32)
            iy = (y * np.float32(RES)).astype(jnp.int32)
            iz = (z * np.float32(RES)).astype(jnp.int32)
            hi = jnp.maximum(jnp.maximum(x, y), z)
            lo = jnp.minimum(jnp.minimum(x, y), z)
            inv = (hi >= HI) | (lo < LO)
            idx = ix * (RES * RES) + iy * RES + iz
            idx_v[s] = jnp.where(inv, SENTINEL, idx)
            return carry

        lax.fori_loop(0, GROUPS, grp, 0, unroll=2)

    def start_gather(b):
        pltpu.async_copy(grid_hbm.at[ib[b]], rb[b], g_sem)

    def finish_gather(j, b):
        pltpu.make_async_copy(grid_hbm.at[ib[b]], rb[b], g_sem).wait()
        pltpu.sync_copy(rb[b], out_hbm.at[pl.ds(slot_base(j), C)])

    start_in(0, 0)

    def pair_body(p, carry):
        j0 = 2 * p
        j1 = 2 * p + 1
        wait_in(0)

        @pl.when(slot_valid(j1))
        def _():
            start_in(j1, 1)

        compute(0)

        @pl.when(p > 0)
        def _():
            finish_gather(j1 - 2, 1)

        start_gather(0)

        @pl.when(slot_valid(j1))
        def _():
            wait_in(1)

        @pl.when(p < N_PAIRS - 1)
        def _():
            start_in(j0 + 2, 0)

        @pl.when(slot_valid(j1))
        def _():
            compute(1)

        finish_gather(j0, 0)

        @pl.when(slot_valid(j1))
        def _():
            start_gather(1)

        return carry

    lax.fori_loop(0, N_PAIRS, pair_body, 0)

    @pl.when(slot_valid(N_SLOTS - 1))
    def _():
        finish_gather(N_SLOTS - 1, 1)


def kernel(pts, grid_flat):
    pts_t = pts.T
    x = pts_t[0]
    y = pts_t[1]
    z = pts_t[2]
    return _occupancy_kernel(x, y, z, grid_flat)
